# baseline (speedup 1.0000x reference)
import functools
import math

import jax
import jax.numpy as jnp
from jax import lax
from jax.experimental import pallas as pl
from jax.experimental.pallas import tpu as pltpu
from jax.experimental.pallas import tpu_sc as plsc

B = 2048
D = 1024
H = 1024
E = 8
K = 2
P = B * K
BMG = 256
NR = P + E * BMG
NBLK = NR // BMG
NT = 16
PT = P // NT
NRT = NR // NT
NW = 32
GR = NR // NW
GC = 64

CNT_OFF = 0
GIX_OFF = 256
ROW_OFF = 256 + NR
WRW_OFF = 256 + 2 * NR
SHM_LEN = 256 + 3 * NR


def _rank_and_last(v):
    cum = jnp.zeros((16,), jnp.int32)
    lastm = jnp.zeros((16,), jnp.bool_)
    for e in range(E):
        m = v == e
        mi = m.astype(jnp.int32)
        ci = plsc.cumsum(mi)
        rc = lax.rev(plsc.cumsum(lax.rev(mi, (0,))), (0,))
        cum = cum + jnp.where(m, ci, 0)
        lastm = lastm | (m & (rc == 1))
    return cum, lastm


def _routing_body(idxf, hwf, x_hbm, wrow, rowp, bexp, xs,
                  shm, idx_v, hw_v, hwi_v, cnt_v, cmat_v, base_v, cumb_v,
                  posb, tokb, pb, pwb, zb_i, zb_f, bexp_b, idxg_v, rows_v, sem):
    c = lax.axis_index("c")
    s = lax.axis_index("s")
    lane = lax.broadcasted_iota(jnp.int32, (16,), 0)

    base = s * PT
    pltpu.sync_copy(idxf.at[pl.ds(base, PT)], idx_v)
    pltpu.sync_copy(hwf.at[pl.ds(base, PT)], hw_v)

    cnt_v[...] = jnp.zeros((16,), jnp.int32)

    def _hist(ch, _):
        v = idx_v[pl.ds(ch * 16, 16)]
        cum, lastm = _rank_and_last(v)
        g0 = plsc.load_gather(cnt_v, [v])
        plsc.store_scatter(cnt_v, [v], g0 + cum, mask=lastm)
        hwi_v[pl.ds(ch * 16, 16)] = plsc.bitcast(hw_v[pl.ds(ch * 16, 16)], jnp.int32)
        return 0

    lax.fori_loop(0, PT // 16, _hist, 0)
    pltpu.sync_copy(cnt_v, shm.at[pl.ds(CNT_OFF + s * 16, 16)])

    def _zfill(ch, _):
        zb_i[pl.ds(ch * 16, 16)] = jnp.zeros((16,), jnp.int32)
        return 0

    lax.fori_loop(0, NRT // 16, _zfill, 0)
    pltpu.sync_copy(zb_i, shm.at[pl.ds(GIX_OFF + s * NRT, NRT)])
    pltpu.sync_copy(zb_i, shm.at[pl.ds(WRW_OFF + s * NRT, NRT)])

    def _zfill2(ch, _):
        zb_i[pl.ds(ch * 16, 16)] = jnp.full((16,), P, jnp.int32)
        return 0

    lax.fori_loop(0, NRT // 16, _zfill2, 0)
    pltpu.sync_copy(zb_i, shm.at[pl.ds(ROW_OFF + s * NRT, NRT)])

    plsc.subcore_barrier()

    pltpu.sync_copy(shm.at[pl.ds(CNT_OFF, NT * 16)], cmat_v)
    total = jnp.zeros((16,), jnp.int32)
    before = jnp.zeros((16,), jnp.int32)
    for t in range(NT):
        row = cmat_v[pl.ds(t * 16, 16)]
        total = total + row
        before = before + row * (s > t).astype(jnp.int32)
    padded = ((total + (BMG - 1)) // BMG) * BMG
    startv = plsc.cumsum(padded) - padded
    base_v[...] = startv + before

    nb_v = padded // BMG
    startb = plsc.cumsum(nb_v) - nb_v
    bexp_b[pl.ds(0, 16)] = jnp.full((16,), -1, jnp.int32)
    bexp_b[pl.ds(16, 16)] = jnp.full((16,), -1, jnp.int32)

    def _bexp(j, _):
        m = (nb_v > j) & (lane < E)
        plsc.store_scatter(bexp_b, [startb + j], lane, mask=m)
        return 0

    lax.fori_loop(0, 16, _bexp, 0)

    @pl.when(s == 0)
    def _():
        pltpu.sync_copy(bexp_b, bexp)

    for g in range(4):
        def _scan(ch2, _):
            ch = g * 4 + ch2
            v = idx_v[pl.ds(ch * 16, 16)]
            p_vec = (base + ch * 16) + lane
            tok = p_vec // K
            cum, lastm = _rank_and_last(v)
            g0 = plsc.load_gather(base_v, [v])
            plsc.store_scatter(base_v, [v], g0 + cum, mask=lastm)
            pos = g0 + cum - 1
            posb[g, pl.ds(ch2 * 16, 16)] = pos + GIX_OFF
            pb[g, pl.ds(ch2 * 16, 16)] = pos + ROW_OFF
            pwb[g, pl.ds(ch2 * 16, 16)] = pos + WRW_OFF
            tokb[g, pl.ds(ch2 * 16, 16)] = tok
            return 0

        lax.fori_loop(0, 4, _scan, 0)

    plsc.subcore_barrier()

    for g in range(4):
        pltpu.sync_copy(tokb.at[g], shm.at[posb.at[g]])
        pltpu.sync_copy(hwi_v.at[pl.ds(g * 64, 64)], shm.at[pwb.at[g]])

    for g in range(4):
        def _pfill(ch2, _):
            tokb[g, pl.ds(ch2 * 16, 16)] = (base + g * 64 + ch2 * 16) + lane
            return 0
        lax.fori_loop(0, 4, _pfill, 0)
        pltpu.sync_copy(tokb.at[g], shm.at[pb.at[g]])

    plsc.subcore_barrier()

    @pl.when(c == 0)
    def _():
        pltpu.sync_copy(shm.at[pl.ds(WRW_OFF + s * NRT, NRT)], zb_i)

        def _bc(ch, _):
            zb_f[pl.ds(ch * 16, 16)] = plsc.bitcast(zb_i[pl.ds(ch * 16, 16)], jnp.float32)
            return 0

        lax.fori_loop(0, NRT // 16, _bc, 0)
        pltpu.sync_copy(zb_f, wrow.at[pl.ds(s * NRT, NRT)])
        pltpu.sync_copy(shm.at[pl.ds(ROW_OFF + s * NRT, NRT)], zb_i)
        pltpu.sync_copy(zb_i, rowp.at[pl.ds(s * NRT, NRT)])

    w = s * 2 + c
    for j in range(GR // GC):
        o = w * GR + j * GC
        pltpu.sync_copy(shm.at[pl.ds(GIX_OFF + o, GC)], idxg_v)
        pltpu.async_copy(x_hbm.at[idxg_v], rows_v, sem).wait()
        pltpu.sync_copy(rows_v, xs.at[pl.ds(o, GC)])


_routing = pl.kernel(
    _routing_body,
    mesh=plsc.VectorSubcoreMesh(core_axis_name="c", subcore_axis_name="s"),
    out_type=[
        jax.ShapeDtypeStruct((NR,), jnp.float32),
        jax.ShapeDtypeStruct((NR,), jnp.int32),
        jax.ShapeDtypeStruct((32,), jnp.int32),
        jax.ShapeDtypeStruct((NR, D), jnp.float32),
    ],
    scratch_types=[
        pltpu.VMEM_SHARED((SHM_LEN,), jnp.int32),
        pltpu.VMEM((PT,), jnp.int32),
        pltpu.VMEM((PT,), jnp.float32),
        pltpu.VMEM((PT,), jnp.int32),
        pltpu.VMEM((16,), jnp.int32),
        pltpu.VMEM((NT * 16,), jnp.int32),
        pltpu.VMEM((16,), jnp.int32),
        pltpu.VMEM((16,), jnp.int32),
        pltpu.VMEM((4, 64), jnp.int32),
        pltpu.VMEM((4, 64), jnp.int32),
        pltpu.VMEM((4, 64), jnp.int32),
        pltpu.VMEM((4, 64), jnp.int32),
        pltpu.VMEM((NRT,), jnp.int32),
        pltpu.VMEM((NRT,), jnp.float32),
        pltpu.VMEM((32,), jnp.int32),
        pltpu.VMEM((GC,), jnp.int32),
        pltpu.VMEM((GC, D), jnp.float32),
        pltpu.SemaphoreType.DMA,
    ],
    compiler_params=pltpu.CompilerParams(needs_layout_passes=False),
)


def _gmm_body(bexp_ref, xs_ref, w_ref, W1_ref, b1_ref, W2_ref, b2_ref, out_ref):
    i = pl.program_id(0)
    be = bexp_ref[i]

    @pl.when(be >= 0)
    def _():
        x = xs_ref[...].astype(jnp.bfloat16)
        h = jnp.dot(x, W1_ref[0].astype(jnp.bfloat16),
                    preferred_element_type=jnp.float32) + b1_ref[0]
        h = 0.5 * h * (1.0 + jax.lax.erf(h * (1.0 / math.sqrt(2.0))))
        y = jnp.dot(h.astype(jnp.bfloat16), W2_ref[0].astype(jnp.bfloat16),
                    preferred_element_type=jnp.float32) + b2_ref[0]
        out_ref[...] = y * w_ref[...]


def _scatter_body(ys, rowp_h, yflat, iv, rv, sem):
    c = lax.axis_index("c")
    s = lax.axis_index("s")
    w = s * 2 + c
    for j in range(GR // GC):
        o = w * GR + j * GC
        pltpu.sync_copy(rowp_h.at[pl.ds(o, GC)], iv)
        pltpu.sync_copy(ys.at[pl.ds(o, GC)], rv)
        pltpu.sync_copy(rv, yflat.at[iv])


_scatter = pl.kernel(
    _scatter_body,
    mesh=plsc.VectorSubcoreMesh(core_axis_name="c", subcore_axis_name="s"),
    out_type=[jax.ShapeDtypeStruct((P + 2, D), jnp.float32)],
    scratch_types=[
        pltpu.VMEM((GC,), jnp.int32),
        pltpu.VMEM((GC, D), jnp.float32),
        pltpu.SemaphoreType.DMA,
    ],
    compiler_params=pltpu.CompilerParams(needs_layout_passes=False),
)


def _comb_body(y_ref, o_ref):
    y = y_ref[...]
    o_ref[...] = y[:, :D] + y[:, D:]


def kernel(x_modality, expert_indices, hard_weights, W1, b1, W2, b2):
    idxf = expert_indices.astype(jnp.int32).reshape(P)
    hwf = hard_weights.reshape(P)
    b1r = b1.reshape(E, 1, H)
    b2r = b2.reshape(E, 1, D)

    wrowv, rowp, bexp, xs = _routing(idxf, hwf, x_modality)

    ys = pl.pallas_call(
        _gmm_body,
        grid_spec=pltpu.PrefetchScalarGridSpec(
            num_scalar_prefetch=1,
            grid=(NBLK,),
            in_specs=[
                pl.BlockSpec((BMG, D), lambda i, be: (i, 0)),
                pl.BlockSpec((BMG, 1), lambda i, be: (i, 0)),
                pl.BlockSpec((1, D, H), lambda i, be: (jnp.maximum(be[i], 0), 0, 0)),
                pl.BlockSpec((1, 1, H), lambda i, be: (jnp.maximum(be[i], 0), 0, 0)),
                pl.BlockSpec((1, H, D), lambda i, be: (jnp.maximum(be[i], 0), 0, 0)),
                pl.BlockSpec((1, 1, D), lambda i, be: (jnp.maximum(be[i], 0), 0, 0)),
            ],
            out_specs=pl.BlockSpec((BMG, D), lambda i, be: (i, 0)),
        ),
        out_shape=jax.ShapeDtypeStruct((NR, D), jnp.float32),
        compiler_params=pltpu.CompilerParams(
            dimension_semantics=("arbitrary",),
        ),
    )(bexp[:NBLK], xs, wrowv.reshape(NR, 1), W1, b1r, W2, b2r)

    yflat = _scatter(ys, rowp)[0]
    yr = yflat.reshape((P + 2) // 2, 2 * D)

    BMC = 256
    return pl.pallas_call(
        _comb_body,
        grid=(B // BMC,),
        in_specs=[pl.BlockSpec((BMC, 2 * D), lambda i: (i, 0))],
        out_specs=pl.BlockSpec((BMC, D), lambda i: (i, 0)),
        out_shape=jax.ShapeDtypeStruct((B, D), jnp.float32),
    )(yr)


# speedup vs baseline: 1.0201x; 1.0201x over previous
import functools
import math

import jax
import jax.numpy as jnp
from jax import lax
from jax.experimental import pallas as pl
from jax.experimental.pallas import tpu as pltpu
from jax.experimental.pallas import tpu_sc as plsc

B = 2048
D = 1024
H = 1024
E = 8
K = 2
P = B * K
BMG = 256
NR = P + E * BMG
NBLK = NR // BMG
NT = 16
PT = P // NT
NRT = NR // NT
NW = 32
GR = NR // NW
GC = 96

CNT_OFF = 0
GIX_OFF = 256
ROW_OFF = 256 + NR
WRW_OFF = 256 + 2 * NR
SHM_LEN = 256 + 3 * NR


def _rank_and_last(v):
    cum = jnp.zeros((16,), jnp.int32)
    lastm = jnp.zeros((16,), jnp.bool_)
    for e in range(E):
        m = v == e
        mi = m.astype(jnp.int32)
        ci = plsc.cumsum(mi)
        rc = lax.rev(plsc.cumsum(lax.rev(mi, (0,))), (0,))
        cum = cum + jnp.where(m, ci, 0)
        lastm = lastm | (m & (rc == 1))
    return cum, lastm


def _routing_body(idxf, hwf, x_hbm, wrow, rowp, bexp, xs,
                  shm, idx_v, hw_v, hwi_v, cnt_v, cmat_v, base_v, cumb_v,
                  posb, tokb, pb, pwb, zb_i, zb_f, bexp_b, idxg_v, rows_v, sem):
    c = lax.axis_index("c")
    s = lax.axis_index("s")
    lane = lax.broadcasted_iota(jnp.int32, (16,), 0)

    base = s * PT
    pltpu.sync_copy(idxf.at[pl.ds(base, PT)], idx_v)
    pltpu.sync_copy(hwf.at[pl.ds(base, PT)], hw_v)

    cnt_v[...] = jnp.zeros((16,), jnp.int32)

    def _hist(ch, _):
        v = idx_v[pl.ds(ch * 16, 16)]
        cum, lastm = _rank_and_last(v)
        g0 = plsc.load_gather(cnt_v, [v])
        plsc.store_scatter(cnt_v, [v], g0 + cum, mask=lastm)
        hwi_v[pl.ds(ch * 16, 16)] = plsc.bitcast(hw_v[pl.ds(ch * 16, 16)], jnp.int32)
        return 0

    lax.fori_loop(0, PT // 16, _hist, 0)
    pltpu.sync_copy(cnt_v, shm.at[pl.ds(CNT_OFF + s * 16, 16)])

    def _zfill(ch, _):
        zb_i[pl.ds(ch * 16, 16)] = jnp.zeros((16,), jnp.int32)
        return 0

    lax.fori_loop(0, NRT // 16, _zfill, 0)
    pltpu.sync_copy(zb_i, shm.at[pl.ds(GIX_OFF + s * NRT, NRT)])
    pltpu.sync_copy(zb_i, shm.at[pl.ds(WRW_OFF + s * NRT, NRT)])

    def _zfill2(ch, _):
        zb_i[pl.ds(ch * 16, 16)] = jnp.full((16,), P, jnp.int32)
        return 0

    lax.fori_loop(0, NRT // 16, _zfill2, 0)
    pltpu.sync_copy(zb_i, shm.at[pl.ds(ROW_OFF + s * NRT, NRT)])

    plsc.subcore_barrier()

    pltpu.sync_copy(shm.at[pl.ds(CNT_OFF, NT * 16)], cmat_v)
    total = jnp.zeros((16,), jnp.int32)
    before = jnp.zeros((16,), jnp.int32)
    for t in range(NT):
        row = cmat_v[pl.ds(t * 16, 16)]
        total = total + row
        before = before + row * (s > t).astype(jnp.int32)
    padded = ((total + (BMG - 1)) // BMG) * BMG
    startv = plsc.cumsum(padded) - padded
    base_v[...] = startv + before

    nb_v = padded // BMG
    startb = plsc.cumsum(nb_v) - nb_v
    bexp_b[pl.ds(0, 16)] = jnp.full((16,), -1, jnp.int32)
    bexp_b[pl.ds(16, 16)] = jnp.full((16,), -1, jnp.int32)

    def _bexp(j, _):
        m = (nb_v > j) & (lane < E)
        plsc.store_scatter(bexp_b, [startb + j], lane, mask=m)
        return 0

    lax.fori_loop(0, 16, _bexp, 0)

    @pl.when(s == 0)
    def _():
        pltpu.sync_copy(bexp_b, bexp)

    for g in range(4):
        def _scan(ch2, _):
            ch = g * 4 + ch2
            v = idx_v[pl.ds(ch * 16, 16)]
            p_vec = (base + ch * 16) + lane
            tok = p_vec // K
            cum, lastm = _rank_and_last(v)
            g0 = plsc.load_gather(base_v, [v])
            plsc.store_scatter(base_v, [v], g0 + cum, mask=lastm)
            pos = g0 + cum - 1
            posb[g, pl.ds(ch2 * 16, 16)] = pos + GIX_OFF
            pb[g, pl.ds(ch2 * 16, 16)] = pos + ROW_OFF
            pwb[g, pl.ds(ch2 * 16, 16)] = pos + WRW_OFF
            tokb[g, pl.ds(ch2 * 16, 16)] = tok
            return 0

        lax.fori_loop(0, 4, _scan, 0)

    plsc.subcore_barrier()

    for g in range(4):
        pltpu.sync_copy(tokb.at[g], shm.at[posb.at[g]])
        pltpu.sync_copy(hwi_v.at[pl.ds(g * 64, 64)], shm.at[pwb.at[g]])

    for g in range(4):
        def _pfill(ch2, _):
            tokb[g, pl.ds(ch2 * 16, 16)] = (base + g * 64 + ch2 * 16) + lane
            return 0
        lax.fori_loop(0, 4, _pfill, 0)
        pltpu.sync_copy(tokb.at[g], shm.at[pb.at[g]])

    plsc.subcore_barrier()

    @pl.when(c == 0)
    def _():
        pltpu.sync_copy(shm.at[pl.ds(WRW_OFF + s * NRT, NRT)], zb_i)

        def _bc(ch, _):
            zb_f[pl.ds(ch * 16, 16)] = plsc.bitcast(zb_i[pl.ds(ch * 16, 16)], jnp.float32)
            return 0

        lax.fori_loop(0, NRT // 16, _bc, 0)
        pltpu.sync_copy(zb_f, wrow.at[pl.ds(s * NRT, NRT)])
        pltpu.sync_copy(shm.at[pl.ds(ROW_OFF + s * NRT, NRT)], zb_i)
        pltpu.sync_copy(zb_i, rowp.at[pl.ds(s * NRT, NRT)])

    w = s * 2 + c
    for j in range(GR // GC):
        o = w * GR + j * GC
        pltpu.sync_copy(shm.at[pl.ds(GIX_OFF + o, GC)], idxg_v)
        pltpu.async_copy(x_hbm.at[idxg_v], rows_v, sem).wait()
        pltpu.sync_copy(rows_v, xs.at[pl.ds(o, GC)])


_routing = pl.kernel(
    _routing_body,
    mesh=plsc.VectorSubcoreMesh(core_axis_name="c", subcore_axis_name="s"),
    out_type=[
        jax.ShapeDtypeStruct((NR,), jnp.float32),
        jax.ShapeDtypeStruct((NR,), jnp.int32),
        jax.ShapeDtypeStruct((32,), jnp.int32),
        jax.ShapeDtypeStruct((NR, D), jnp.float32),
    ],
    scratch_types=[
        pltpu.VMEM_SHARED((SHM_LEN,), jnp.int32),
        pltpu.VMEM((PT,), jnp.int32),
        pltpu.VMEM((PT,), jnp.float32),
        pltpu.VMEM((PT,), jnp.int32),
        pltpu.VMEM((16,), jnp.int32),
        pltpu.VMEM((NT * 16,), jnp.int32),
        pltpu.VMEM((16,), jnp.int32),
        pltpu.VMEM((16,), jnp.int32),
        pltpu.VMEM((4, 64), jnp.int32),
        pltpu.VMEM((4, 64), jnp.int32),
        pltpu.VMEM((4, 64), jnp.int32),
        pltpu.VMEM((4, 64), jnp.int32),
        pltpu.VMEM((NRT,), jnp.int32),
        pltpu.VMEM((NRT,), jnp.float32),
        pltpu.VMEM((32,), jnp.int32),
        pltpu.VMEM((GC,), jnp.int32),
        pltpu.VMEM((GC, D), jnp.float32),
        pltpu.SemaphoreType.DMA,
    ],
    compiler_params=pltpu.CompilerParams(needs_layout_passes=False),
)


def _gmm_body(bexp_ref, xs_ref, w_ref, W1_ref, b1_ref, W2_ref, b2_ref, out_ref):
    i = pl.program_id(0)
    be = bexp_ref[i]

    @pl.when(be >= 0)
    def _():
        x = xs_ref[...].astype(jnp.bfloat16)
        h = jnp.dot(x, W1_ref[0].astype(jnp.bfloat16),
                    preferred_element_type=jnp.float32) + b1_ref[0]
        h = 0.5 * h * (1.0 + jax.lax.erf(h * (1.0 / math.sqrt(2.0))))
        y = jnp.dot(h.astype(jnp.bfloat16), W2_ref[0].astype(jnp.bfloat16),
                    preferred_element_type=jnp.float32) + b2_ref[0]
        out_ref[...] = y * w_ref[...]


def _scatter_body(ys, rowp_h, yflat, iv, rv, sem):
    c = lax.axis_index("c")
    s = lax.axis_index("s")
    w = s * 2 + c
    for j in range(GR // GC):
        o = w * GR + j * GC
        pltpu.sync_copy(rowp_h.at[pl.ds(o, GC)], iv)
        pltpu.sync_copy(ys.at[pl.ds(o, GC)], rv)
        pltpu.sync_copy(rv, yflat.at[iv])


_scatter = pl.kernel(
    _scatter_body,
    mesh=plsc.VectorSubcoreMesh(core_axis_name="c", subcore_axis_name="s"),
    out_type=[jax.ShapeDtypeStruct((P + 2, D), jnp.float32)],
    scratch_types=[
        pltpu.VMEM((GC,), jnp.int32),
        pltpu.VMEM((GC, D), jnp.float32),
        pltpu.SemaphoreType.DMA,
    ],
    compiler_params=pltpu.CompilerParams(needs_layout_passes=False),
)


def _comb_body(y_ref, o_ref):
    y = y_ref[...]
    o_ref[...] = y[:, :D] + y[:, D:]


def kernel(x_modality, expert_indices, hard_weights, W1, b1, W2, b2):
    idxf = expert_indices.astype(jnp.int32).reshape(P)
    hwf = hard_weights.reshape(P)
    b1r = b1.reshape(E, 1, H)
    b2r = b2.reshape(E, 1, D)

    wrowv, rowp, bexp, xs = _routing(idxf, hwf, x_modality)

    ys = pl.pallas_call(
        _gmm_body,
        grid_spec=pltpu.PrefetchScalarGridSpec(
            num_scalar_prefetch=1,
            grid=(NBLK,),
            in_specs=[
                pl.BlockSpec((BMG, D), lambda i, be: (i, 0)),
                pl.BlockSpec((BMG, 1), lambda i, be: (i, 0)),
                pl.BlockSpec((1, D, H), lambda i, be: (jnp.maximum(be[i], 0), 0, 0)),
                pl.BlockSpec((1, 1, H), lambda i, be: (jnp.maximum(be[i], 0), 0, 0)),
                pl.BlockSpec((1, H, D), lambda i, be: (jnp.maximum(be[i], 0), 0, 0)),
                pl.BlockSpec((1, 1, D), lambda i, be: (jnp.maximum(be[i], 0), 0, 0)),
            ],
            out_specs=pl.BlockSpec((BMG, D), lambda i, be: (i, 0)),
        ),
        out_shape=jax.ShapeDtypeStruct((NR, D), jnp.float32),
        compiler_params=pltpu.CompilerParams(
            dimension_semantics=("arbitrary",),
        ),
    )(bexp[:NBLK], xs, wrowv.reshape(NR, 1), W1, b1r, W2, b2r)

    yflat = _scatter(ys, rowp)[0]
    yr = yflat.reshape((P + 2) // 2, 2 * D)

    BMC = 256
    return pl.pallas_call(
        _comb_body,
        grid=(B // BMC,),
        in_specs=[pl.BlockSpec((BMC, 2 * D), lambda i: (i, 0))],
        out_specs=pl.BlockSpec((BMC, D), lambda i: (i, 0)),
        out_shape=jax.ShapeDtypeStruct((B, D), jnp.float32),
    )(yr)
